# Initial kernel scaffold; baseline (speedup 1.0000x reference)
import jax
import jax.numpy as jnp
from jax.experimental import pallas as pl


def kernel(x, edge_index, edge_index_pairs, W1, b1, W2, b2, Wlin, blin):
    raise NotImplementedError("write your pallas kernel here")


# speedup vs baseline: 14.0744x; 14.0744x over previous
import functools

import jax
import jax.numpy as jnp
from jax import lax
from jax.experimental import pallas as pl
from jax.experimental.pallas import tpu as pltpu
from jax.experimental.pallas import tpu_sc as plsc

N = 10000
D = 128
H = 128
E = 320000
P = 100000

NC = 2
NS = 16
NW = NC * NS
CW = 128

EC = 80
EPW = EC * CW
PAD_E = NW * EPW

GC = 25
PPW = GC * CW
PAD_P = NW * PPW

NROW = 10112
RPT = NROW // NS
DN = 10112
DPT = DN // NS

_MESH = plsc.VectorSubcoreMesh(
    core_axis_name="c", subcore_axis_name="s", num_cores=NC, num_subcores=NS)


def _deg_body(dstp_hbm, ones_hbm, zdeg_hbm, degp_hbm, dst_v, buf_v, ones_v,
              acc_sh):
    c = lax.axis_index("c")
    s = lax.axis_index("s")
    w = s * NC + c
    pltpu.sync_copy(dstp_hbm.at[w], dst_v)
    pltpu.sync_copy(ones_hbm, ones_v)
    pltpu.sync_copy(zdeg_hbm, buf_v)
    pltpu.sync_copy(buf_v, acc_sh.at[pl.ds(s * DPT, DPT)])
    plsc.subcore_barrier()

    def chunk(g, carry):
        pltpu.sync_copy(ones_v, acc_sh.at[dst_v.at[g]], add=True)
        return carry

    lax.fori_loop(0, EC, chunk, 0)
    plsc.subcore_barrier()
    pltpu.sync_copy(acc_sh.at[pl.ds(s * DPT, DPT)], buf_v)
    pltpu.sync_copy(buf_v, degp_hbm.at[pl.ds(c * DN + s * DPT, DPT)])


_deg_call = pl.kernel(
    _deg_body,
    out_type=jax.ShapeDtypeStruct((NC * DN,), jnp.float32),
    mesh=_MESH,
    scratch_types=[
        pltpu.VMEM((EC, CW), jnp.int32),
        pltpu.VMEM((DPT,), jnp.float32),
        pltpu.VMEM((CW,), jnp.float32),
        pltpu.VMEM_SHARED((DN,), jnp.float32),
    ],
)


def _scatter_body(a_hbm, srcp_hbm, dstp_hbm, zrow_hbm, out_hbm, src_v, dst_v,
                  rbuf, acc_sh):
    c = lax.axis_index("c")
    s = lax.axis_index("s")
    w = s * NC + c
    pltpu.sync_copy(srcp_hbm.at[w], src_v)
    pltpu.sync_copy(dstp_hbm.at[w], dst_v)
    base = s * RPT
    rem = RPT - 4 * CW
    pltpu.sync_copy(zrow_hbm, rbuf)
    for k in range(4):
        pltpu.sync_copy(rbuf, acc_sh.at[pl.ds(base + k * CW, CW)])
    pltpu.sync_copy(rbuf.at[pl.ds(0, rem)],
                    acc_sh.at[pl.ds(base + 4 * CW, rem)])
    plsc.subcore_barrier()

    def chunk(g, carry):
        pltpu.sync_copy(a_hbm.at[src_v.at[g]], rbuf)
        pltpu.sync_copy(rbuf, acc_sh.at[dst_v.at[g]], add=True)
        return carry

    lax.fori_loop(0, EC, chunk, 0)
    plsc.subcore_barrier()
    for k in range(4):
        pltpu.sync_copy(acc_sh.at[pl.ds(base + k * CW, CW)], rbuf)
        pltpu.sync_copy(rbuf, out_hbm.at[c, pl.ds(base + k * CW, CW)])
    pltpu.sync_copy(acc_sh.at[pl.ds(base + 4 * CW, rem)],
                    rbuf.at[pl.ds(0, rem)])
    pltpu.sync_copy(rbuf.at[pl.ds(0, rem)],
                    out_hbm.at[c, pl.ds(base + 4 * CW, rem)])


_scatter_call = pl.kernel(
    _scatter_body,
    out_type=jax.ShapeDtypeStruct((NC, NROW, D), jnp.float32),
    mesh=_MESH,
    scratch_types=[
        pltpu.VMEM((EC, CW), jnp.int32),
        pltpu.VMEM((EC, CW), jnp.int32),
        pltpu.VMEM((CW, D), jnp.float32),
        pltpu.VMEM_SHARED((NROW, D), jnp.float32),
    ],
)


TW = 16


def _decode_body(t_hbm, ipp_hbm, jpp_hbm, ou_hbm, ov_hbm, ip_v, jp_v,
                 gbuf, gbuf2, obu, obv):
    c = lax.axis_index("c")
    s = lax.axis_index("s")
    w = s * NC + c
    pltpu.sync_copy(ipp_hbm.at[w], ip_v)
    pltpu.sync_copy(jpp_hbm.at[w], jp_v)

    def chunk(g, carry):
        orow = pl.ds(w * PPW + g * CW, CW)
        pltpu.sync_copy(t_hbm.at[ip_v.at[g]], gbuf)
        pltpu.sync_copy(t_hbm.at[jp_v.at[g]], gbuf2)

        def repack(r, carry2):
            obu[r] = gbuf[r, pl.ds(0, TW)]
            obv[r] = gbuf2[r, pl.ds(0, TW)]
            return carry2

        lax.fori_loop(0, CW, repack, 0)
        pltpu.sync_copy(obu, ou_hbm.at[orow])
        pltpu.sync_copy(obv, ov_hbm.at[orow])
        return carry

    lax.fori_loop(0, GC, chunk, 0)


_decode_call = pl.kernel(
    _decode_body,
    out_type=[jax.ShapeDtypeStruct((PAD_P, TW), jnp.float32)] * 2,
    mesh=_MESH,
    scratch_types=[
        pltpu.VMEM((GC, CW), jnp.int32),
        pltpu.VMEM((GC, CW), jnp.int32),
        pltpu.VMEM((CW, D), jnp.float32),
        pltpu.VMEM((CW, D), jnp.float32),
        pltpu.VMEM((CW, TW), jnp.float32),
        pltpu.VMEM((CW, TW), jnp.float32),
    ],
)


_DOT = functools.partial(jnp.dot, preferred_element_type=jnp.float32,
                         precision=lax.Precision.HIGHEST)


def _dinv_of(dg_ref):
    deg = dg_ref[0] + dg_ref[1] + 1.0
    return lax.rsqrt(jnp.maximum(deg, 1.0))


def _tc_a(x_ref, w1_ref, dg_ref, xw1_ref, a1_ref):
    dinv = _dinv_of(dg_ref)
    xw = _DOT(x_ref[...], w1_ref[...])
    xw1_ref[...] = xw
    a1_ref[...] = xw * dinv


def _tc_c(s1_ref, xw1_ref, dg_ref, b1_ref, w2_ref, xw2_ref, a2_ref):
    dinv = _dinv_of(dg_ref)
    ssum = s1_ref[0] + s1_ref[1]
    h = dinv * ssum + (dinv * dinv) * xw1_ref[...] + b1_ref[...]
    h = jnp.maximum(h, 0.0)
    xw2 = _DOT(h, w2_ref[...])
    xw2_ref[...] = xw2
    a2_ref[...] = xw2 * dinv


def _tc_d(s2_ref, xw2_ref, dg_ref, b2_ref, wt_ref, bt_ref, t_ref):
    dinv = _dinv_of(dg_ref)
    ssum = s2_ref[0] + s2_ref[1]
    z = dinv * ssum + (dinv * dinv) * xw2_ref[...] + b2_ref[...]
    t_ref[...] = _DOT(z, wt_ref[...]) + bt_ref[...]


def _tc_e(ou_ref, ov_ref, o_ref):
    o_ref[...] = ou_ref[:, 0:2] + ov_ref[:, 2:4]


def kernel(x, edge_index, edge_index_pairs, W1, b1, W2, b2, Wlin, blin):
    f32 = jnp.float32
    x = x.astype(f32)

    epad = PAD_E - E
    src = jnp.concatenate(
        [edge_index[0], jnp.arange(epad, dtype=jnp.int32) % N])
    dst = jnp.concatenate(
        [edge_index[1], jnp.full((epad,), N, jnp.int32)])
    srcp = src.reshape(NW, EC, CW)
    dstp = dst.reshape(NW, EC, CW)

    ppad = PAD_P - P
    ipp = jnp.concatenate(
        [edge_index_pairs[0], jnp.arange(ppad, dtype=jnp.int32) % N]
    ).reshape(NW, GC, CW)
    jpp = jnp.concatenate(
        [edge_index_pairs[1], jnp.arange(ppad, dtype=jnp.int32) % N]
    ).reshape(NW, GC, CW)

    ones = jnp.ones((CW,), f32)
    zdeg = jnp.zeros((DPT,), f32)
    zrow = jnp.zeros((CW, D), f32)

    wt = jnp.zeros((H, D), f32).at[:, 0:2].set(Wlin[:H]).at[:, 2:4].set(
        Wlin[H:])
    bt = jnp.zeros((D,), f32).at[:2].set(blin)

    degp = _deg_call(dstp, ones, zdeg)
    dg = degp.reshape(NC, DN)[:, :N].reshape(NC, N, 1)

    BR = 1000
    _row = pl.BlockSpec((BR, D), lambda i: (i, 0))
    _prt = pl.BlockSpec((2, BR, D), lambda i: (0, i, 0))
    _dgb = pl.BlockSpec((2, BR, 1), lambda i: (0, i, 0))
    _vec = pl.BlockSpec((D,), lambda i: (0,))
    _mat = pl.BlockSpec((D, D), lambda i: (0, 0))

    xw1, a1 = pl.pallas_call(
        _tc_a,
        grid=(N // BR,),
        in_specs=[_row, _mat, _dgb],
        out_specs=[_row, _row],
        out_shape=[jax.ShapeDtypeStruct((N, D), f32)] * 2,
    )(x, W1, dg)

    s1p = _scatter_call(a1, srcp, dstp, zrow)

    xw2, a2 = pl.pallas_call(
        _tc_c,
        grid=(N // BR,),
        in_specs=[_prt, _row, _dgb, _vec, _mat],
        out_specs=[_row, _row],
        out_shape=[jax.ShapeDtypeStruct((N, H), f32)] * 2,
    )(s1p, xw1, dg, b1, W2)

    s2p = _scatter_call(a2, srcp, dstp, zrow)

    t = pl.pallas_call(
        _tc_d,
        grid=(DN // BR + 1,),
        in_specs=[_prt, _row, _dgb, _vec, _mat, _vec],
        out_specs=_row,
        out_shape=jax.ShapeDtypeStruct((DN, D), f32),
    )(s2p, xw2, dg, b2, wt, bt)

    ou, ov = _decode_call(t, ipp, jpp)

    BP = 1024
    oadd = pl.pallas_call(
        _tc_e,
        grid=(PAD_P // BP,),
        in_specs=[pl.BlockSpec((BP, TW), lambda i: (i, 0))] * 2,
        out_specs=pl.BlockSpec((BP, 2), lambda i: (i, 0)),
        out_shape=jax.ShapeDtypeStruct((PAD_P, 2), f32),
    )(ou, ov)
    return oadd[:P]


# speedup vs baseline: 16.9626x; 1.2052x over previous
import functools

import jax
import jax.numpy as jnp
from jax import lax
from jax.experimental import pallas as pl
from jax.experimental.pallas import tpu as pltpu
from jax.experimental.pallas import tpu_sc as plsc

N = 10000
D = 128
H = 128
E = 320000
P = 100000

NC = 2
NS = 16
NW = NC * NS
CW = 128

EC = 80
IB = 40
EPW = EC * CW
PAD_E = NW * EPW

GC = 25
PPW = GC * CW
PAD_P = NW * PPW

NROW = 10112
RPT = NROW // NS
DN = 10112
DPT = DN // NS

_MESH = plsc.VectorSubcoreMesh(
    core_axis_name="c", subcore_axis_name="s", num_cores=NC, num_subcores=NS)


def _deg_body(dstp_hbm, ones_hbm, zdeg_hbm, degp_hbm, dst_v, buf_v, ones_v,
              acc_sh):
    c = lax.axis_index("c")
    s = lax.axis_index("s")
    w = s * NC + c
    pltpu.sync_copy(ones_hbm, ones_v)
    pltpu.sync_copy(zdeg_hbm, buf_v)
    pltpu.sync_copy(buf_v, acc_sh.at[pl.ds(s * DPT, DPT)])
    plsc.subcore_barrier()

    for hb in range(EC // IB):
        pltpu.sync_copy(dstp_hbm.at[w * (EC // IB) + hb], dst_v)

        def chunk(g, carry):
            pltpu.sync_copy(ones_v, acc_sh.at[dst_v.at[g]], add=True)
            return carry

        lax.fori_loop(0, IB, chunk, 0)
    plsc.subcore_barrier()
    pltpu.sync_copy(acc_sh.at[pl.ds(s * DPT, DPT)], buf_v)
    pltpu.sync_copy(buf_v, degp_hbm.at[pl.ds(c * DN + s * DPT, DPT)])


_deg_call = pl.kernel(
    _deg_body,
    out_type=jax.ShapeDtypeStruct((NC * DN,), jnp.float32),
    mesh=_MESH,
    scratch_types=[
        pltpu.VMEM((IB, CW), jnp.int32),
        pltpu.VMEM((DPT,), jnp.float32),
        pltpu.VMEM((CW,), jnp.float32),
        pltpu.VMEM_SHARED((DN,), jnp.float32),
    ],
)


def _scatter_body(a_hbm, srcp_hbm, dstp_hbm, zrow_hbm, out_hbm, src_v, dst_v,
                  rbuf0, rbuf1, acc_sh, sem0, sem1):
    c = lax.axis_index("c")
    s = lax.axis_index("s")
    w = s * NC + c
    pltpu.sync_copy(srcp_hbm.at[w], src_v)
    pltpu.sync_copy(dstp_hbm.at[w], dst_v)
    base = s * RPT
    rem = RPT - 4 * CW
    pltpu.sync_copy(zrow_hbm, rbuf0)
    for k in range(4):
        pltpu.sync_copy(rbuf0, acc_sh.at[pl.ds(base + k * CW, CW)])
    pltpu.sync_copy(rbuf0.at[pl.ds(0, rem)],
                    acc_sh.at[pl.ds(base + 4 * CW, rem)])
    plsc.subcore_barrier()

    for hb in range(EC // IB):
        pltpu.sync_copy(srcp_hbm.at[w * (EC // IB) + hb], src_v)
        pltpu.sync_copy(dstp_hbm.at[w * (EC // IB) + hb], dst_v)
        pltpu.async_copy(a_hbm.at[src_v.at[0]], rbuf0, sem0)
        pltpu.async_copy(a_hbm.at[src_v.at[1]], rbuf1, sem1)

        def body(i, carry):
            g0 = 2 * i
            pltpu.make_async_copy(zrow_hbm, rbuf0, sem0).wait()
            pltpu.sync_copy(rbuf0, acc_sh.at[dst_v.at[g0]], add=True)
            pltpu.async_copy(a_hbm.at[src_v.at[jnp.minimum(g0 + 2, IB - 2)]],
                             rbuf0, sem0)
            g1 = g0 + 1
            pltpu.make_async_copy(zrow_hbm, rbuf1, sem1).wait()
            pltpu.sync_copy(rbuf1, acc_sh.at[dst_v.at[g1]], add=True)
            pltpu.async_copy(a_hbm.at[src_v.at[jnp.minimum(g1 + 2, IB - 1)]],
                             rbuf1, sem1)
            return carry

        lax.fori_loop(0, IB // 2, body, 0)
        pltpu.make_async_copy(zrow_hbm, rbuf0, sem0).wait()
        pltpu.make_async_copy(zrow_hbm, rbuf1, sem1).wait()
    plsc.subcore_barrier()
    for k in range(4):
        pltpu.sync_copy(acc_sh.at[pl.ds(base + k * CW, CW)], rbuf0)
        pltpu.sync_copy(rbuf0, out_hbm.at[c, pl.ds(base + k * CW, CW)])
    pltpu.sync_copy(acc_sh.at[pl.ds(base + 4 * CW, rem)],
                    rbuf0.at[pl.ds(0, rem)])
    pltpu.sync_copy(rbuf0.at[pl.ds(0, rem)],
                    out_hbm.at[c, pl.ds(base + 4 * CW, rem)])


_scatter_call = pl.kernel(
    _scatter_body,
    out_type=jax.ShapeDtypeStruct((NC, NROW, D), jnp.float32),
    mesh=_MESH,
    scratch_types=[
        pltpu.VMEM((IB, CW), jnp.int32),
        pltpu.VMEM((IB, CW), jnp.int32),
        pltpu.VMEM((CW, D), jnp.float32),
        pltpu.VMEM((CW, D), jnp.float32),
        pltpu.VMEM_SHARED((NROW, D), jnp.float32),
        pltpu.SemaphoreType.DMA,
        pltpu.SemaphoreType.DMA,
    ],
)


TW = 16


def _decode_body(t_hbm, ipp_hbm, jpp_hbm, ou_hbm, ov_hbm, ip_v, jp_v,
                 gbuf, gbuf2, obu, obv):
    c = lax.axis_index("c")
    s = lax.axis_index("s")
    w = s * NC + c
    pltpu.sync_copy(ipp_hbm.at[w], ip_v)
    pltpu.sync_copy(jpp_hbm.at[w], jp_v)

    def chunk(g, carry):
        orow = pl.ds(w * PPW + g * CW, CW)
        pltpu.sync_copy(t_hbm.at[ip_v.at[g]], gbuf)
        pltpu.sync_copy(t_hbm.at[jp_v.at[g]], gbuf2)

        def repack(r, carry2):
            obu[r] = gbuf[r, pl.ds(0, TW)]
            obv[r] = gbuf2[r, pl.ds(0, TW)]
            return carry2

        lax.fori_loop(0, CW, repack, 0)
        pltpu.sync_copy(obu, ou_hbm.at[orow])
        pltpu.sync_copy(obv, ov_hbm.at[orow])
        return carry

    lax.fori_loop(0, GC, chunk, 0)


_decode_call = pl.kernel(
    _decode_body,
    out_type=[jax.ShapeDtypeStruct((PAD_P, TW), jnp.float32)] * 2,
    mesh=_MESH,
    scratch_types=[
        pltpu.VMEM((GC, CW), jnp.int32),
        pltpu.VMEM((GC, CW), jnp.int32),
        pltpu.VMEM((CW, D), jnp.float32),
        pltpu.VMEM((CW, D), jnp.float32),
        pltpu.VMEM((CW, TW), jnp.float32),
        pltpu.VMEM((CW, TW), jnp.float32),
    ],
)


_DOT = functools.partial(jnp.dot, preferred_element_type=jnp.float32,
                         precision=lax.Precision.HIGHEST)


def _dinv_of(dg_ref):
    deg = dg_ref[0] + dg_ref[1] + 1.0
    return lax.rsqrt(jnp.maximum(deg, 1.0))


def _tc_a(x_ref, w1_ref, dg_ref, a1_ref):
    dinv = _dinv_of(dg_ref)
    a1_ref[...] = _DOT(x_ref[...], w1_ref[...]) * dinv


def _tc_c(s1_ref, a1_ref, dg_ref, b1_ref, w2_ref, a2_ref):
    dinv = _dinv_of(dg_ref)
    h = dinv * (s1_ref[0] + s1_ref[1] + a1_ref[...]) + b1_ref[...]
    h = jnp.maximum(h, 0.0)
    a2_ref[...] = _DOT(h, w2_ref[...]) * dinv


def _tc_d(s2_ref, a2_ref, dg_ref, b2_ref, wt_ref, bt_ref, t_ref):
    dinv = _dinv_of(dg_ref)
    z = dinv * (s2_ref[0] + s2_ref[1] + a2_ref[...]) + b2_ref[...]
    t_ref[...] = _DOT(z, wt_ref[...]) + bt_ref[...]


def _tc_e(ou_ref, ov_ref, o_ref):
    o_ref[...] = ou_ref[:, 0:2] + ov_ref[:, 2:4]


def kernel(x, edge_index, edge_index_pairs, W1, b1, W2, b2, Wlin, blin):
    f32 = jnp.float32
    x = x.astype(f32)

    epad = PAD_E - E
    src = jnp.concatenate(
        [edge_index[0], jnp.arange(epad, dtype=jnp.int32) % N])
    dst = jnp.concatenate(
        [edge_index[1], jnp.full((epad,), N, jnp.int32)])
    srcp = src.reshape(NW * (EC // IB), IB, CW)
    dstp = dst.reshape(NW * (EC // IB), IB, CW)

    ppad = PAD_P - P
    ipp = jnp.concatenate(
        [edge_index_pairs[0], jnp.arange(ppad, dtype=jnp.int32) % N]
    ).reshape(NW, GC, CW)
    jpp = jnp.concatenate(
        [edge_index_pairs[1], jnp.arange(ppad, dtype=jnp.int32) % N]
    ).reshape(NW, GC, CW)

    ones = jnp.ones((CW,), f32)
    zdeg = jnp.zeros((DPT,), f32)
    zrow = jnp.zeros((CW, D), f32)

    wt = jnp.zeros((H, D), f32).at[:, 0:2].set(Wlin[:H]).at[:, 2:4].set(
        Wlin[H:])
    bt = jnp.zeros((D,), f32).at[:2].set(blin)

    degp = _deg_call(dstp, ones, zdeg)
    dg = degp.reshape(NC, DN)[:, :N].reshape(NC, N, 1)

    BR = 1000
    _row = pl.BlockSpec((BR, D), lambda i: (i, 0))
    _prt = pl.BlockSpec((2, BR, D), lambda i: (0, i, 0))
    _dgb = pl.BlockSpec((2, BR, 1), lambda i: (0, i, 0))
    _vec = pl.BlockSpec((D,), lambda i: (0,))
    _mat = pl.BlockSpec((D, D), lambda i: (0, 0))

    a1 = pl.pallas_call(
        _tc_a,
        grid=(N // BR,),
        in_specs=[_row, _mat, _dgb],
        out_specs=_row,
        out_shape=jax.ShapeDtypeStruct((N, D), f32),
    )(x, W1, dg)

    s1p = _scatter_call(a1, srcp, dstp, zrow)

    a2 = pl.pallas_call(
        _tc_c,
        grid=(N // BR,),
        in_specs=[_prt, _row, _dgb, _vec, _mat],
        out_specs=_row,
        out_shape=jax.ShapeDtypeStruct((N, H), f32),
    )(s1p, a1, dg, b1, W2)

    s2p = _scatter_call(a2, srcp, dstp, zrow)

    t = pl.pallas_call(
        _tc_d,
        grid=(DN // BR + 1,),
        in_specs=[_prt, _row, _dgb, _vec, _mat, _vec],
        out_specs=_row,
        out_shape=jax.ShapeDtypeStruct((DN, D), f32),
    )(s2p, a2, dg, b2, wt, bt)

    ou, ov = _decode_call(t, ipp, jpp)

    BP = 1024
    oadd = pl.pallas_call(
        _tc_e,
        grid=(PAD_P // BP,),
        in_specs=[pl.BlockSpec((BP, TW), lambda i: (i, 0))] * 2,
        out_specs=pl.BlockSpec((BP, 2), lambda i: (i, 0)),
        out_shape=jax.ShapeDtypeStruct((PAD_P, 2), f32),
    )(ou, ov)
    return oadd[:P]


# speedup vs baseline: 17.7811x; 1.0483x over previous
import functools

import jax
import jax.numpy as jnp
from jax import lax
from jax.experimental import pallas as pl
from jax.experimental.pallas import tpu as pltpu
from jax.experimental.pallas import tpu_sc as plsc

N = 10000
D = 128
H = 128
E = 320000
P = 100000

NC = 2
NS = 16
NW = NC * NS
CW = 128

EC = 80
IB = 40
EPW = EC * CW
PAD_E = NW * EPW

GC = 26
PPW = GC * CW
PAD_P = NW * PPW

NROW = 10112
RPT = NROW // NS
DN = 10112
DPT = DN // NS

_MESH = plsc.VectorSubcoreMesh(
    core_axis_name="c", subcore_axis_name="s", num_cores=NC, num_subcores=NS)


def _deg_body(dstp_hbm, ones_hbm, zdeg_hbm, degp_hbm, dst_v, buf_v, ones_v,
              acc_sh):
    c = lax.axis_index("c")
    s = lax.axis_index("s")
    w = s * NC + c
    pltpu.sync_copy(ones_hbm, ones_v)
    pltpu.sync_copy(zdeg_hbm, buf_v)
    pltpu.sync_copy(buf_v, acc_sh.at[pl.ds(s * DPT, DPT)])
    plsc.subcore_barrier()

    for hb in range(EC // IB):
        pltpu.sync_copy(dstp_hbm.at[w * (EC // IB) + hb], dst_v)

        def chunk(g, carry):
            pltpu.sync_copy(ones_v, acc_sh.at[dst_v.at[g]], add=True)
            return carry

        lax.fori_loop(0, IB, chunk, 0)
    plsc.subcore_barrier()
    pltpu.sync_copy(acc_sh.at[pl.ds(s * DPT, DPT)], buf_v)
    pltpu.sync_copy(buf_v, degp_hbm.at[pl.ds(c * DN + s * DPT, DPT)])


_deg_call = pl.kernel(
    _deg_body,
    out_type=jax.ShapeDtypeStruct((NC * DN,), jnp.float32),
    mesh=_MESH,
    scratch_types=[
        pltpu.VMEM((IB, CW), jnp.int32),
        pltpu.VMEM((DPT,), jnp.float32),
        pltpu.VMEM((CW,), jnp.float32),
        pltpu.VMEM_SHARED((DN,), jnp.float32),
    ],
)


def _scatter_body(a_hbm, srcp_hbm, dstp_hbm, zrow_hbm, out_hbm, src_v, dst_v,
                  rbuf0, rbuf1, acc_sh, sem0, sem1):
    c = lax.axis_index("c")
    s = lax.axis_index("s")
    w = s * NC + c
    pltpu.sync_copy(srcp_hbm.at[w], src_v)
    pltpu.sync_copy(dstp_hbm.at[w], dst_v)
    base = s * RPT
    rem = RPT - 4 * CW
    pltpu.sync_copy(zrow_hbm, rbuf0)
    for k in range(4):
        pltpu.sync_copy(rbuf0, acc_sh.at[pl.ds(base + k * CW, CW)])
    pltpu.sync_copy(rbuf0.at[pl.ds(0, rem)],
                    acc_sh.at[pl.ds(base + 4 * CW, rem)])
    plsc.subcore_barrier()

    for hb in range(EC // IB):
        pltpu.sync_copy(srcp_hbm.at[w * (EC // IB) + hb], src_v)
        pltpu.sync_copy(dstp_hbm.at[w * (EC // IB) + hb], dst_v)
        pltpu.async_copy(a_hbm.at[src_v.at[0]], rbuf0, sem0)
        pltpu.async_copy(a_hbm.at[src_v.at[1]], rbuf1, sem1)

        def body(i, carry):
            g0 = 2 * i
            pltpu.make_async_copy(zrow_hbm, rbuf0, sem0).wait()
            pltpu.sync_copy(rbuf0, acc_sh.at[dst_v.at[g0]], add=True)
            pltpu.async_copy(a_hbm.at[src_v.at[jnp.minimum(g0 + 2, IB - 2)]],
                             rbuf0, sem0)
            g1 = g0 + 1
            pltpu.make_async_copy(zrow_hbm, rbuf1, sem1).wait()
            pltpu.sync_copy(rbuf1, acc_sh.at[dst_v.at[g1]], add=True)
            pltpu.async_copy(a_hbm.at[src_v.at[jnp.minimum(g1 + 2, IB - 1)]],
                             rbuf1, sem1)
            return carry

        lax.fori_loop(0, IB // 2, body, 0)
        pltpu.make_async_copy(zrow_hbm, rbuf0, sem0).wait()
        pltpu.make_async_copy(zrow_hbm, rbuf1, sem1).wait()
    plsc.subcore_barrier()
    for k in range(4):
        pltpu.sync_copy(acc_sh.at[pl.ds(base + k * CW, CW)], rbuf0)
        pltpu.sync_copy(rbuf0, out_hbm.at[c, pl.ds(base + k * CW, CW)])
    pltpu.sync_copy(acc_sh.at[pl.ds(base + 4 * CW, rem)],
                    rbuf0.at[pl.ds(0, rem)])
    pltpu.sync_copy(rbuf0.at[pl.ds(0, rem)],
                    out_hbm.at[c, pl.ds(base + 4 * CW, rem)])


_scatter_call = pl.kernel(
    _scatter_body,
    out_type=jax.ShapeDtypeStruct((NC, NROW, D), jnp.float32),
    mesh=_MESH,
    scratch_types=[
        pltpu.VMEM((IB, CW), jnp.int32),
        pltpu.VMEM((IB, CW), jnp.int32),
        pltpu.VMEM((CW, D), jnp.float32),
        pltpu.VMEM((CW, D), jnp.float32),
        pltpu.VMEM_SHARED((NROW, D), jnp.float32),
        pltpu.SemaphoreType.DMA,
        pltpu.SemaphoreType.DMA,
    ],
)


TW = 16


def _decode_body(t_hbm, ipp_hbm, jpp_hbm, ou_hbm, ov_hbm, ip_v, jp_v,
                 ga0, gb0, ga1, gb1, obu, obv, sem0, sem1):
    c = lax.axis_index("c")
    s = lax.axis_index("s")
    w = s * NC + c
    pltpu.sync_copy(ipp_hbm.at[w], ip_v)
    pltpu.sync_copy(jpp_hbm.at[w], jp_v)
    dummy = t_hbm.at[pl.ds(0, CW)]

    pltpu.async_copy(t_hbm.at[ip_v.at[0]], ga0, sem0)
    pltpu.async_copy(t_hbm.at[jp_v.at[0]], gb0, sem0)
    pltpu.async_copy(t_hbm.at[ip_v.at[1]], ga1, sem1)
    pltpu.async_copy(t_hbm.at[jp_v.at[1]], gb1, sem1)

    def emit(g, ga, gb):
        def repack(r, carry2):
            obu[r] = ga[r, pl.ds(0, TW)]
            obv[r] = gb[r, pl.ds(0, TW)]
            return carry2

        lax.fori_loop(0, CW, repack, 0)
        orow = pl.ds(w * PPW + g * CW, CW)
        pltpu.sync_copy(obu, ou_hbm.at[orow])
        pltpu.sync_copy(obv, ov_hbm.at[orow])

    def body(i, carry):
        g0 = 2 * i
        pltpu.make_async_copy(dummy, ga0, sem0).wait()
        pltpu.make_async_copy(dummy, gb0, sem0).wait()
        emit(g0, ga0, gb0)
        nxt = jnp.minimum(g0 + 2, GC - 2)
        pltpu.async_copy(t_hbm.at[ip_v.at[nxt]], ga0, sem0)
        pltpu.async_copy(t_hbm.at[jp_v.at[nxt]], gb0, sem0)
        g1 = g0 + 1
        pltpu.make_async_copy(dummy, ga1, sem1).wait()
        pltpu.make_async_copy(dummy, gb1, sem1).wait()
        emit(g1, ga1, gb1)
        nxt1 = jnp.minimum(g1 + 2, GC - 1)
        pltpu.async_copy(t_hbm.at[ip_v.at[nxt1]], ga1, sem1)
        pltpu.async_copy(t_hbm.at[jp_v.at[nxt1]], gb1, sem1)
        return carry

    lax.fori_loop(0, GC // 2, body, 0)
    pltpu.make_async_copy(dummy, ga0, sem0).wait()
    pltpu.make_async_copy(dummy, gb0, sem0).wait()
    pltpu.make_async_copy(dummy, ga1, sem1).wait()
    pltpu.make_async_copy(dummy, gb1, sem1).wait()


_decode_call = pl.kernel(
    _decode_body,
    out_type=[jax.ShapeDtypeStruct((PAD_P, TW), jnp.float32)] * 2,
    mesh=_MESH,
    scratch_types=[
        pltpu.VMEM((GC, CW), jnp.int32),
        pltpu.VMEM((GC, CW), jnp.int32),
        pltpu.VMEM((CW, D), jnp.float32),
        pltpu.VMEM((CW, D), jnp.float32),
        pltpu.VMEM((CW, D), jnp.float32),
        pltpu.VMEM((CW, D), jnp.float32),
        pltpu.VMEM((CW, TW), jnp.float32),
        pltpu.VMEM((CW, TW), jnp.float32),
        pltpu.SemaphoreType.DMA,
        pltpu.SemaphoreType.DMA,
    ],
)


_DOT = functools.partial(jnp.dot, preferred_element_type=jnp.float32,
                         precision=lax.Precision.HIGHEST)


def _dinv_of(dg_ref):
    deg = dg_ref[0] + dg_ref[1] + 1.0
    return lax.rsqrt(jnp.maximum(deg, 1.0))


def _tc_a(x_ref, w1_ref, dg_ref, a1_ref):
    dinv = _dinv_of(dg_ref)
    a1_ref[...] = _DOT(x_ref[...], w1_ref[...]) * dinv


def _tc_c(s1_ref, a1_ref, dg_ref, b1_ref, w2_ref, a2_ref):
    dinv = _dinv_of(dg_ref)
    h = dinv * (s1_ref[0] + s1_ref[1] + a1_ref[...]) + b1_ref[...]
    h = jnp.maximum(h, 0.0)
    a2_ref[...] = _DOT(h, w2_ref[...]) * dinv


def _tc_d(s2_ref, a2_ref, dg_ref, b2_ref, wt_ref, bt_ref, t_ref):
    dinv = _dinv_of(dg_ref)
    z = dinv * (s2_ref[0] + s2_ref[1] + a2_ref[...]) + b2_ref[...]
    t_ref[...] = _DOT(z, wt_ref[...]) + bt_ref[...]


def _tc_e(ou_ref, ov_ref, o_ref):
    o_ref[...] = ou_ref[:, 0:2] + ov_ref[:, 2:4]


def kernel(x, edge_index, edge_index_pairs, W1, b1, W2, b2, Wlin, blin):
    f32 = jnp.float32
    x = x.astype(f32)

    epad = PAD_E - E
    src = jnp.concatenate(
        [edge_index[0], jnp.arange(epad, dtype=jnp.int32) % N])
    dst = jnp.concatenate(
        [edge_index[1], jnp.full((epad,), N, jnp.int32)])
    srcp = src.reshape(NW * (EC // IB), IB, CW)
    dstp = dst.reshape(NW * (EC // IB), IB, CW)

    ppad = PAD_P - P
    ipp = jnp.concatenate(
        [edge_index_pairs[0], jnp.arange(ppad, dtype=jnp.int32) % N]
    ).reshape(NW, GC, CW)
    jpp = jnp.concatenate(
        [edge_index_pairs[1], jnp.arange(ppad, dtype=jnp.int32) % N]
    ).reshape(NW, GC, CW)

    ones = jnp.ones((CW,), f32)
    zdeg = jnp.zeros((DPT,), f32)
    zrow = jnp.zeros((CW, D), f32)

    wt = jnp.zeros((H, D), f32).at[:, 0:2].set(Wlin[:H]).at[:, 2:4].set(
        Wlin[H:])
    bt = jnp.zeros((D,), f32).at[:2].set(blin)

    degp = _deg_call(dstp, ones, zdeg)
    dg = degp.reshape(NC, DN)[:, :N].reshape(NC, N, 1)

    BR = 1000
    _row = pl.BlockSpec((BR, D), lambda i: (i, 0))
    _prt = pl.BlockSpec((2, BR, D), lambda i: (0, i, 0))
    _dgb = pl.BlockSpec((2, BR, 1), lambda i: (0, i, 0))
    _vec = pl.BlockSpec((D,), lambda i: (0,))
    _mat = pl.BlockSpec((D, D), lambda i: (0, 0))

    a1 = pl.pallas_call(
        _tc_a,
        grid=(N // BR,),
        in_specs=[_row, _mat, _dgb],
        out_specs=_row,
        out_shape=jax.ShapeDtypeStruct((N, D), f32),
    )(x, W1, dg)

    s1p = _scatter_call(a1, srcp, dstp, zrow)

    a2 = pl.pallas_call(
        _tc_c,
        grid=(N // BR,),
        in_specs=[_prt, _row, _dgb, _vec, _mat],
        out_specs=_row,
        out_shape=jax.ShapeDtypeStruct((N, H), f32),
    )(s1p, a1, dg, b1, W2)

    s2p = _scatter_call(a2, srcp, dstp, zrow)

    t = pl.pallas_call(
        _tc_d,
        grid=(DN // BR + 1,),
        in_specs=[_prt, _row, _dgb, _vec, _mat, _vec],
        out_specs=_row,
        out_shape=jax.ShapeDtypeStruct((DN, D), f32),
    )(s2p, a2, dg, b2, wt, bt)

    ou, ov = _decode_call(t, ipp, jpp)

    BP = 1024
    oadd = pl.pallas_call(
        _tc_e,
        grid=(PAD_P // BP,),
        in_specs=[pl.BlockSpec((BP, TW), lambda i: (i, 0))] * 2,
        out_specs=pl.BlockSpec((BP, 2), lambda i: (i, 0)),
        out_shape=jax.ShapeDtypeStruct((PAD_P, 2), f32),
    )(ou, ov)
    return oadd[:P]


# speedup vs baseline: 22.0481x; 1.2400x over previous
import functools

import jax
import jax.numpy as jnp
from jax import lax
from jax.experimental import pallas as pl
from jax.experimental.pallas import tpu as pltpu
from jax.experimental.pallas import tpu_sc as plsc

N = 10000
D = 128
H = 128
E = 320000
P = 100000

NC = 2
NS = 16
NW = NC * NS
CW = 128

EC = 80
IB = 40
EPW = EC * CW
PAD_E = NW * EPW

GC = 26
PPW = GC * CW
PAD_P = NW * PPW

NROW = 10112
RPT = NROW // NS
DN = 10112
DPT = DN // NS

_MESH = plsc.VectorSubcoreMesh(
    core_axis_name="c", subcore_axis_name="s", num_cores=NC, num_subcores=NS)


def _deg_body(dstp_hbm, ones_hbm, zdeg_hbm, degp_hbm, dst_v, buf_v, ones_v,
              acc_sh):
    c = lax.axis_index("c")
    s = lax.axis_index("s")
    w = s * NC + c
    pltpu.sync_copy(ones_hbm, ones_v)
    pltpu.sync_copy(zdeg_hbm, buf_v)
    pltpu.sync_copy(buf_v, acc_sh.at[pl.ds(s * DPT, DPT)])
    plsc.subcore_barrier()

    for hb in range(EC // IB):
        pltpu.sync_copy(dstp_hbm.at[w * (EC // IB) + hb], dst_v)

        def chunk(g, carry):
            pltpu.sync_copy(ones_v, acc_sh.at[dst_v.at[g]], add=True)
            return carry

        lax.fori_loop(0, IB, chunk, 0)
    plsc.subcore_barrier()
    pltpu.sync_copy(acc_sh.at[pl.ds(s * DPT, DPT)], buf_v)
    pltpu.sync_copy(buf_v, degp_hbm.at[pl.ds(c * DN + s * DPT, DPT)])


_deg_call = pl.kernel(
    _deg_body,
    out_type=jax.ShapeDtypeStruct((NC * DN,), jnp.float32),
    mesh=_MESH,
    scratch_types=[
        pltpu.VMEM((IB, CW), jnp.int32),
        pltpu.VMEM((DPT,), jnp.float32),
        pltpu.VMEM((CW,), jnp.float32),
        pltpu.VMEM_SHARED((DN,), jnp.float32),
    ],
)


def _scatter_body(a_hbm, srcp_hbm, dstp_hbm, zrow_hbm, out_hbm, src_v, dst_v,
                  rbuf0, rbuf1, acc_sh, sem0, sem1):
    c = lax.axis_index("c")
    s = lax.axis_index("s")
    w = s * NC + c
    pltpu.sync_copy(srcp_hbm.at[w], src_v)
    pltpu.sync_copy(dstp_hbm.at[w], dst_v)
    base = s * RPT
    rem = RPT - 4 * CW
    pltpu.sync_copy(zrow_hbm, rbuf0)
    for k in range(4):
        pltpu.sync_copy(rbuf0, acc_sh.at[pl.ds(base + k * CW, CW)])
    pltpu.sync_copy(rbuf0.at[pl.ds(0, rem)],
                    acc_sh.at[pl.ds(base + 4 * CW, rem)])
    plsc.subcore_barrier()

    for hb in range(EC // IB):
        pltpu.sync_copy(srcp_hbm.at[w * (EC // IB) + hb], src_v)
        pltpu.sync_copy(dstp_hbm.at[w * (EC // IB) + hb], dst_v)
        pltpu.async_copy(a_hbm.at[src_v.at[0]], rbuf0, sem0)
        pltpu.async_copy(a_hbm.at[src_v.at[1]], rbuf1, sem1)

        def body(i, carry):
            g0 = 2 * i
            pltpu.make_async_copy(zrow_hbm, rbuf0, sem0).wait()
            pltpu.sync_copy(rbuf0, acc_sh.at[dst_v.at[g0]], add=True)
            pltpu.async_copy(a_hbm.at[src_v.at[jnp.minimum(g0 + 2, IB - 2)]],
                             rbuf0, sem0)
            g1 = g0 + 1
            pltpu.make_async_copy(zrow_hbm, rbuf1, sem1).wait()
            pltpu.sync_copy(rbuf1, acc_sh.at[dst_v.at[g1]], add=True)
            pltpu.async_copy(a_hbm.at[src_v.at[jnp.minimum(g1 + 2, IB - 1)]],
                             rbuf1, sem1)
            return carry

        lax.fori_loop(0, IB // 2, body, 0)
        pltpu.make_async_copy(zrow_hbm, rbuf0, sem0).wait()
        pltpu.make_async_copy(zrow_hbm, rbuf1, sem1).wait()
    plsc.subcore_barrier()
    for k in range(4):
        pltpu.sync_copy(acc_sh.at[pl.ds(base + k * CW, CW)], rbuf0)
        pltpu.sync_copy(rbuf0, out_hbm.at[c, pl.ds(base + k * CW, CW)])
    pltpu.sync_copy(acc_sh.at[pl.ds(base + 4 * CW, rem)],
                    rbuf0.at[pl.ds(0, rem)])
    pltpu.sync_copy(rbuf0.at[pl.ds(0, rem)],
                    out_hbm.at[c, pl.ds(base + 4 * CW, rem)])


_scatter_call = pl.kernel(
    _scatter_body,
    out_type=jax.ShapeDtypeStruct((NC, NROW, D), jnp.float32),
    mesh=_MESH,
    scratch_types=[
        pltpu.VMEM((IB, CW), jnp.int32),
        pltpu.VMEM((IB, CW), jnp.int32),
        pltpu.VMEM((CW, D), jnp.float32),
        pltpu.VMEM((CW, D), jnp.float32),
        pltpu.VMEM_SHARED((NROW, D), jnp.float32),
        pltpu.SemaphoreType.DMA,
        pltpu.SemaphoreType.DMA,
    ],
)


TW = 16


def _decode_body(u_hbm, v_hbm, ipp_hbm, jpp_hbm, og_hbm, ip_v, jp_v,
                 ga0, gb0, ga1, gb1, obu, sem0, sem1):
    c = lax.axis_index("c")
    s = lax.axis_index("s")
    w = s * NC + c
    pltpu.sync_copy(ipp_hbm.at[w], ip_v)
    pltpu.sync_copy(jpp_hbm.at[w], jp_v)
    dummy = u_hbm.at[pl.ds(0, CW)]

    pltpu.async_copy(u_hbm.at[ip_v.at[0]], ga0, sem0)
    pltpu.async_copy(v_hbm.at[jp_v.at[0]], gb0, sem0)
    pltpu.async_copy(u_hbm.at[ip_v.at[1]], ga1, sem1)
    pltpu.async_copy(v_hbm.at[jp_v.at[1]], gb1, sem1)

    def emit(g, ga, gb):
        def repack(r, carry2):
            obu[r] = ga[r, pl.ds(0, TW)] + gb[r, pl.ds(0, TW)]
            return carry2

        lax.fori_loop(0, CW, repack, 0)
        orow = pl.ds(w * PPW + g * CW, CW)
        pltpu.sync_copy(obu, og_hbm.at[orow])

    def body(i, carry):
        g0 = 2 * i
        pltpu.make_async_copy(dummy, ga0, sem0).wait()
        pltpu.make_async_copy(dummy, gb0, sem0).wait()
        emit(g0, ga0, gb0)
        nxt = jnp.minimum(g0 + 2, GC - 2)
        pltpu.async_copy(u_hbm.at[ip_v.at[nxt]], ga0, sem0)
        pltpu.async_copy(v_hbm.at[jp_v.at[nxt]], gb0, sem0)
        g1 = g0 + 1
        pltpu.make_async_copy(dummy, ga1, sem1).wait()
        pltpu.make_async_copy(dummy, gb1, sem1).wait()
        emit(g1, ga1, gb1)
        nxt1 = jnp.minimum(g1 + 2, GC - 1)
        pltpu.async_copy(u_hbm.at[ip_v.at[nxt1]], ga1, sem1)
        pltpu.async_copy(v_hbm.at[jp_v.at[nxt1]], gb1, sem1)
        return carry

    lax.fori_loop(0, GC // 2, body, 0)
    pltpu.make_async_copy(dummy, ga0, sem0).wait()
    pltpu.make_async_copy(dummy, gb0, sem0).wait()
    pltpu.make_async_copy(dummy, ga1, sem1).wait()
    pltpu.make_async_copy(dummy, gb1, sem1).wait()


_decode_call = pl.kernel(
    _decode_body,
    out_type=jax.ShapeDtypeStruct((PAD_P, TW), jnp.float32),
    mesh=_MESH,
    scratch_types=[
        pltpu.VMEM((GC, CW), jnp.int32),
        pltpu.VMEM((GC, CW), jnp.int32),
        pltpu.VMEM((CW, D), jnp.float32),
        pltpu.VMEM((CW, D), jnp.float32),
        pltpu.VMEM((CW, D), jnp.float32),
        pltpu.VMEM((CW, D), jnp.float32),
        pltpu.VMEM((CW, TW), jnp.float32),
        pltpu.SemaphoreType.DMA,
        pltpu.SemaphoreType.DMA,
    ],
)


_DOT = functools.partial(jnp.dot, preferred_element_type=jnp.float32,
                         precision=lax.Precision.HIGHEST)


def _dinv_of(dg_ref):
    deg = dg_ref[0] + dg_ref[1] + 1.0
    return lax.rsqrt(jnp.maximum(deg, 1.0))


def _tc_a(x_ref, w1_ref, dg_ref, a1_ref):
    dinv = _dinv_of(dg_ref)
    a1_ref[...] = _DOT(x_ref[...], w1_ref[...]) * dinv


def _tc_c(s1_ref, a1_ref, dg_ref, b1_ref, w2_ref, a2_ref):
    dinv = _dinv_of(dg_ref)
    h = dinv * (s1_ref[0] + s1_ref[1] + a1_ref[...]) + b1_ref[...]
    h = jnp.maximum(h, 0.0)
    a2_ref[...] = _DOT(h, w2_ref[...]) * dinv


def _tc_d(s2_ref, a2_ref, dg_ref, b2_ref, wu_ref, wv_ref, bt_ref, u_ref,
          v_ref):
    dinv = _dinv_of(dg_ref)
    z = dinv * (s2_ref[0] + s2_ref[1] + a2_ref[...]) + b2_ref[...]
    u_ref[...] = _DOT(z, wu_ref[...]) + bt_ref[...]
    v_ref[...] = _DOT(z, wv_ref[...])


def kernel(x, edge_index, edge_index_pairs, W1, b1, W2, b2, Wlin, blin):
    f32 = jnp.float32
    x = x.astype(f32)

    epad = PAD_E - E
    src = jnp.concatenate(
        [edge_index[0], jnp.arange(epad, dtype=jnp.int32) % N])
    dst = jnp.concatenate(
        [edge_index[1], jnp.full((epad,), N, jnp.int32)])
    srcp = src.reshape(NW * (EC // IB), IB, CW)
    dstp = dst.reshape(NW * (EC // IB), IB, CW)

    ppad = PAD_P - P
    ipp = jnp.concatenate(
        [edge_index_pairs[0], jnp.arange(ppad, dtype=jnp.int32) % N]
    ).reshape(NW, GC, CW)
    jpp = jnp.concatenate(
        [edge_index_pairs[1], jnp.arange(ppad, dtype=jnp.int32) % N]
    ).reshape(NW, GC, CW)

    ones = jnp.ones((CW,), f32)
    zdeg = jnp.zeros((DPT,), f32)
    zrow = jnp.zeros((CW, D), f32)

    wu = jnp.zeros((H, D), f32).at[:, 0:2].set(Wlin[:H])
    wv = jnp.zeros((H, D), f32).at[:, 0:2].set(Wlin[H:])
    bt = jnp.zeros((D,), f32).at[:2].set(blin)

    degp = _deg_call(dstp, ones, zdeg)
    dg = degp.reshape(NC, DN)[:, :N].reshape(NC, N, 1)

    BR = 1000
    _row = pl.BlockSpec((BR, D), lambda i: (i, 0))
    _prt = pl.BlockSpec((2, BR, D), lambda i: (0, i, 0))
    _dgb = pl.BlockSpec((2, BR, 1), lambda i: (0, i, 0))
    _vec = pl.BlockSpec((D,), lambda i: (0,))
    _mat = pl.BlockSpec((D, D), lambda i: (0, 0))

    a1 = pl.pallas_call(
        _tc_a,
        grid=(N // BR,),
        in_specs=[_row, _mat, _dgb],
        out_specs=_row,
        out_shape=jax.ShapeDtypeStruct((N, D), f32),
    )(x, W1, dg)

    s1p = _scatter_call(a1, srcp, dstp, zrow)

    a2 = pl.pallas_call(
        _tc_c,
        grid=(N // BR,),
        in_specs=[_prt, _row, _dgb, _vec, _mat],
        out_specs=_row,
        out_shape=jax.ShapeDtypeStruct((N, H), f32),
    )(s1p, a1, dg, b1, W2)

    s2p = _scatter_call(a2, srcp, dstp, zrow)

    ut, vt = pl.pallas_call(
        _tc_d,
        grid=(DN // BR + 1,),
        in_specs=[_prt, _row, _dgb, _vec, _mat, _mat, _vec],
        out_specs=[_row, _row],
        out_shape=[jax.ShapeDtypeStruct((DN, D), f32)] * 2,
    )(s2p, a2, dg, b2, wu, wv, bt)

    og = _decode_call(ut, vt, ipp, jpp)
    return og[:P, :2]
